# Initial kernel scaffold; baseline (speedup 1.0000x reference)
import jax
import jax.numpy as jnp
from jax.experimental import pallas as pl


def kernel(x, expert_bias, W, eps):
    raise NotImplementedError("write your pallas kernel here")


# speedup vs baseline: 1.0631x; 1.0631x over previous
import functools

import jax
import jax.numpy as jnp
from jax import lax
from jax.experimental import pallas as pl
from jax.experimental.pallas import tpu as pltpu
from jax.experimental.pallas import tpu_sc as plsc

TOKENS = 16384
HIDDEN = 2048
EXPERTS = 16
TOPK = 2

NC = 2
NS = 16
NW = NC * NS
TPW = TOKENS // NW
GROUPS = TPW // 16

LN2 = 0.6931471805599453


def _scores_body(x_ref, wt_ref, out_ref):
    z = jnp.dot(x_ref[...], wt_ref[...], preferred_element_type=jnp.float32)
    out_ref[...] = 1.0 / (1.0 + jnp.exp(-z))


def _tc_scores(x, Wt):
    BT = 1024
    return pl.pallas_call(
        _scores_body,
        grid=(TOKENS // BT,),
        in_specs=[
            pl.BlockSpec((BT, HIDDEN), lambda i: (i, 0)),
            pl.BlockSpec((HIDDEN, EXPERTS), lambda i: (0, 0)),
        ],
        out_specs=pl.BlockSpec((BT, EXPERTS), lambda i: (i, 0)),
        out_shape=jax.ShapeDtypeStruct((TOKENS, EXPERTS), jnp.float32),
    )(x, Wt)


def _ln(x):
    bi = lax.bitcast_convert_type(x, jnp.int32)
    e = lax.shift_right_arithmetic(bi, 23) - 127
    mb = lax.bitwise_or(lax.bitwise_and(bi, 0x7FFFFF), 0x3F800000)
    m = lax.bitcast_convert_type(mb, jnp.float32)
    t = (m - 1.0) / (m + 1.0)
    t2 = t * t
    ln_m = t * (2.0 + t2 * (2.0 / 3.0 + t2 * (2.0 / 5.0 + t2 * (2.0 / 7.0))))
    return e.astype(jnp.float32) * LN2 + ln_m


_sc_mesh = plsc.VectorSubcoreMesh(
    core_axis_name="c", subcore_axis_name="s", num_cores=NC, num_subcores=NS)


_SC_KERNEL_KWARGS = dict(
    out_type=(
        jax.ShapeDtypeStruct((TOKENS * TOPK,), jnp.float32),
        jax.ShapeDtypeStruct((TOKENS * TOPK,), jnp.int32),
        jax.ShapeDtypeStruct((NC * 16,), jnp.float32),
        jax.ShapeDtypeStruct((NC * 16,), jnp.float32),
    ),
    mesh=_sc_mesh,
    compiler_params=pltpu.CompilerParams(needs_layout_passes=False),
    scratch_types=[
        pltpu.VMEM((TPW * EXPERTS,), jnp.float32),
        pltpu.VMEM((TPW * TOPK,), jnp.float32),
        pltpu.VMEM((TPW * TOPK,), jnp.int32),
        pltpu.VMEM((EXPERTS,), jnp.float32),
        pltpu.VMEM((16,), jnp.float32),
        pltpu.VMEM((16 * EXPERTS,), jnp.float32),
        pltpu.VMEM((16,), jnp.float32),
        pltpu.VMEM((16,), jnp.float32),
        pltpu.VMEM((2 * NS * 16,), jnp.float32),
        pltpu.VMEM_SHARED((2 * NS * 16,), jnp.float32),
    ],
)


def _sc_route_body(scores_hbm, bias_hbm, eps_hbm,
              top_hbm, sel_hbm, cnt_hbm, ent_hbm,
              scores_v, top_v, sel_v, bias_v, eps_v, hist_v, cnt_v, ent_v,
              stage_v, shared):
    cid = lax.axis_index("c")
    sid = lax.axis_index("s")
    wid = sid * NC + cid
    base = wid * TPW
    pltpu.sync_copy(scores_hbm.at[pl.ds(base * EXPERTS, TPW * EXPERTS)],
                    scores_v)
    pltpu.sync_copy(bias_hbm, bias_v)
    pltpu.sync_copy(eps_hbm, eps_v)

    zeros16 = jnp.zeros((16,), jnp.float32)
    for l in range(16):
        hist_v[pl.ds(l * 16, 16)] = zeros16
    ent_v[...] = zeros16

    lanes = lax.iota(jnp.int32, 16)
    ones_f = jnp.ones((16,), jnp.float32)
    big = jnp.full((16,), EXPERTS, jnp.int32)
    neg_inf = jnp.full((16,), -jnp.inf, jnp.float32)
    eps_s = eps_v[...][0]
    bias_vec = bias_v[...]

    @pl.loop(0, GROUPS)
    def body(g):
        flat0 = (g * 16 + lanes) * EXPERTS
        b = []
        for e in range(EXPERTS):
            v = plsc.load_gather(scores_v, [flat0 + e])
            b.append(v + bias_vec[e])
        m1 = functools.reduce(jnp.maximum, b)
        idx1 = functools.reduce(
            jnp.minimum,
            [jnp.where(b[e] == m1, jnp.full((16,), e, jnp.int32), big)
             for e in range(EXPERTS)])
        s1 = m1 - plsc.load_gather(bias_v, [idx1])
        b2 = [jnp.where(idx1 == e, neg_inf, b[e]) for e in range(EXPERTS)]
        m2 = functools.reduce(jnp.maximum, b2)
        idx2 = functools.reduce(
            jnp.minimum,
            [jnp.where(b2[e] == m2, jnp.full((16,), e, jnp.int32), big)
             for e in range(EXPERTS)])
        s2 = m2 - plsc.load_gather(bias_v, [idx2])
        r = 1.0 / (s1 + s2 + eps_s)
        t1 = s1 * r
        t2 = s2 * r
        plsc.addupdate(ent_v.at[...], -(t1 * _ln(t1) + t2 * _ln(t2)))
        plsc.addupdate_scatter(hist_v, [lanes * EXPERTS + idx1], ones_f)
        plsc.addupdate_scatter(hist_v, [lanes * EXPERTS + idx2], ones_f)
        out0 = (g * 16 + lanes) * TOPK
        plsc.store_scatter(top_v, [out0], t1)
        plsc.store_scatter(top_v, [out0 + 1], t2)
        plsc.store_scatter(sel_v, [out0], idx1)
        plsc.store_scatter(sel_v, [out0 + 1], idx2)

    pltpu.sync_copy(top_v, top_hbm.at[pl.ds(base * TOPK, TPW * TOPK)])
    pltpu.sync_copy(sel_v, sel_hbm.at[pl.ds(base * TOPK, TPW * TOPK)])

    cnt = hist_v[pl.ds(0, 16)]
    for l in range(1, 16):
        cnt = cnt + hist_v[pl.ds(l * 16, 16)]
    cnt_v[...] = cnt
    pltpu.sync_copy(cnt_v, shared.at[pl.ds(sid * 16, 16)])
    pltpu.sync_copy(ent_v, shared.at[pl.ds((NS + sid) * 16, 16)])
    plsc.subcore_barrier()

    @pl.when(sid == 0)
    def _():
        pltpu.sync_copy(shared, stage_v)
        cacc = stage_v[pl.ds(0, 16)]
        for i in range(1, NS):
            cacc = cacc + stage_v[pl.ds(i * 16, 16)]
        eacc = stage_v[pl.ds(NS * 16, 16)]
        for i in range(1, NS):
            eacc = eacc + stage_v[pl.ds((NS + i) * 16, 16)]
        cnt_v[...] = cacc
        ent_v[...] = eacc
        pltpu.sync_copy(cnt_v, cnt_hbm.at[pl.ds(cid * 16, 16)])
        pltpu.sync_copy(ent_v, ent_hbm.at[pl.ds(cid * 16, 16)])


_sc_route = pl.kernel(_sc_route_body, **_SC_KERNEL_KWARGS)


def _combine_body(cnt_part_ref, ent_part_ref, cnt_ref, ent_ref):
    cnt_ref[...] = jnp.sum(cnt_part_ref[...], axis=0, keepdims=True)
    ent_ref[...] = jnp.broadcast_to(
        jnp.sum(ent_part_ref[...]) * (1.0 / TOKENS), (1, 16))


def _tc_combine(cnt_part, ent_part):
    return pl.pallas_call(
        _combine_body,
        out_shape=(
            jax.ShapeDtypeStruct((1, 16), jnp.float32),
            jax.ShapeDtypeStruct((1, 16), jnp.float32),
        ),
    )(cnt_part, ent_part)


def kernel(x, expert_bias, W, eps):
    scores = _tc_scores(x, W.T)
    eps16 = jnp.full((16,), eps, jnp.float32)
    top_flat, sel_flat, cnt_part, ent_part = _sc_route(
        scores.reshape(-1), expert_bias, eps16)
    counts2, ent2 = _tc_combine(cnt_part.reshape(NC, 16),
                                ent_part.reshape(NC, 16))
    top_scores = top_flat.reshape(TOKENS, TOPK)
    sel_idx = sel_flat.reshape(TOKENS, TOPK)
    return top_scores, scores, sel_idx, counts2[0], ent2[0, 0]
